# baseline (speedup 1.0000x reference)
import jax
import jax.numpy as jnp
from jax import lax
from jax.experimental import pallas as pl
from jax.experimental.pallas import tpu as pltpu
from jax.experimental.pallas import tpu_sc as plsc

VOCAB = 512
SEQ = 50
BATCH = 1024
A0 = 1.0 / 100.0
A1 = 39.0 / 100.0
A2 = 6.0 / 10.0

NC = 2
NS = 16
L = 16
NW = NC * NS
BPW = BATCH // NW
D = VOCAB
NCHUNK = D // L
OUT_GROUPS = 4
ROWS_PER_GROUP = BPW // OUT_GROUPS


def _body(tokens_hbm, uni_hbm, bi_hbm, tri_hbm, out_hbm,
          prev_v, last_v, idx_v, uni_v, bi_v, tri_v,
          bi_sem, tri_sem, out_sem):
    wid = lax.axis_index("s") * NC + lax.axis_index("c")
    base = wid * BPW

    pltpu.sync_copy(tokens_hbm.at[SEQ - 2, pl.ds(base, BPW)], prev_v)
    pltpu.sync_copy(tokens_hbm.at[SEQ - 1, pl.ds(base, BPW)], last_v)

    for c in range(BPW // L):
        sl = pl.ds(c * L, L)
        idx_v[sl] = prev_v[sl] * VOCAB + last_v[sl]

    bi_copy = pltpu.make_async_copy(bi_hbm.at[last_v], bi_v, bi_sem)
    bi_copy.start()
    tri_copy = pltpu.make_async_copy(tri_hbm.at[idx_v], tri_v, tri_sem)
    tri_copy.start()

    pltpu.sync_copy(uni_hbm, uni_v)
    uni_c = [uni_v[pl.ds(c * L, L)] * A0 for c in range(NCHUNK)]

    bi_copy.wait()
    tri_copy.wait()

    def row(r, carry):
        for c in range(NCHUNK):
            sl = pl.ds(c * L, L)
            bi_v[r, sl] = uni_c[c] + A1 * bi_v[r, sl] + A2 * tri_v[r, sl]
        return carry

    out_copies = []
    for g in range(OUT_GROUPS):
        r0 = g * ROWS_PER_GROUP
        lax.fori_loop(r0, r0 + ROWS_PER_GROUP, row, 0, unroll=2)
        cp = pltpu.make_async_copy(
            bi_v.at[pl.ds(r0, ROWS_PER_GROUP)],
            out_hbm.at[pl.ds(base + r0, ROWS_PER_GROUP)],
            out_sem,
        )
        cp.start()
        out_copies.append(cp)
    for cp in out_copies:
        cp.wait()


@jax.jit
def kernel(input_data, unigram_probs, bigram_probs, trigram_probs):
    tri2d = trigram_probs.reshape(VOCAB * VOCAB, VOCAB)

    mesh = plsc.VectorSubcoreMesh(
        core_axis_name="c", subcore_axis_name="s",
        num_cores=NC, num_subcores=NS,
    )
    run = pl.kernel(
        _body,
        out_type=jax.ShapeDtypeStruct((BATCH, D), jnp.float32),
        mesh=mesh,
        scratch_types=[
            pltpu.VMEM((BPW,), jnp.int32),
            pltpu.VMEM((BPW,), jnp.int32),
            pltpu.VMEM((BPW,), jnp.int32),
            pltpu.VMEM((D,), jnp.float32),
            pltpu.VMEM((BPW, D), jnp.float32),
            pltpu.VMEM((BPW, D), jnp.float32),
            pltpu.SemaphoreType.DMA,
            pltpu.SemaphoreType.DMA,
            pltpu.SemaphoreType.DMA,
        ],
    )
    return run(input_data, unigram_probs, bigram_probs, tri2d)


# speedup vs baseline: 1.2271x; 1.2271x over previous
import jax
import jax.numpy as jnp
from jax import lax
from jax.experimental import pallas as pl
from jax.experimental.pallas import tpu as pltpu
from jax.experimental.pallas import tpu_sc as plsc

VOCAB = 512
SEQ = 50
BATCH = 1024
A0 = 1.0 / 100.0
A1 = 39.0 / 100.0
A2 = 6.0 / 10.0

NC = 2
NS = 16
L = 16
NW = NC * NS
BPW = BATCH // NW
D = VOCAB
NCHUNK = D // L
OUT_GROUPS = 4
ROWS_PER_GROUP = BPW // OUT_GROUPS


def _body(tokens_hbm, uni_hbm, bi_hbm, tri_hbm, out_hbm,
          prev_v, last_v, idx_v, uni_v, bi_v, tri_v,
          bi_sem, tri_sem, out_sem):
    wid = lax.axis_index("s") * NC + lax.axis_index("c")
    base = wid * BPW

    pltpu.sync_copy(tokens_hbm.at[SEQ - 2, pl.ds(base, BPW)], prev_v)
    pltpu.sync_copy(tokens_hbm.at[SEQ - 1, pl.ds(base, BPW)], last_v)

    for c in range(BPW // L):
        sl = pl.ds(c * L, L)
        idx_v[sl] = prev_v[sl] * VOCAB + last_v[sl]

    bi_copy = pltpu.make_async_copy(bi_hbm.at[last_v], bi_v, bi_sem)
    bi_copy.start()
    tri_copy = pltpu.make_async_copy(tri_hbm.at[idx_v], tri_v, tri_sem)
    tri_copy.start()

    pltpu.sync_copy(uni_hbm, uni_v)
    uni_c = [uni_v[pl.ds(c * L, L)] * A0 for c in range(NCHUNK)]

    bi_copy.wait()
    tri_copy.wait()

    out_copies = []
    for g in range(OUT_GROUPS):
        r0 = g * ROWS_PER_GROUP

        @plsc.parallel_loop(r0, r0 + ROWS_PER_GROUP)
        def _blend_rows(r):
            for c in range(NCHUNK):
                sl = pl.ds(c * L, L)
                bi_v[r, sl] = uni_c[c] + A1 * bi_v[r, sl] + A2 * tri_v[r, sl]
        cp = pltpu.make_async_copy(
            bi_v.at[pl.ds(r0, ROWS_PER_GROUP)],
            out_hbm.at[pl.ds(base + r0, ROWS_PER_GROUP)],
            out_sem,
        )
        cp.start()
        out_copies.append(cp)
    for cp in out_copies:
        cp.wait()


@jax.jit
def kernel(input_data, unigram_probs, bigram_probs, trigram_probs):
    tri2d = trigram_probs.reshape(VOCAB * VOCAB, VOCAB)

    mesh = plsc.VectorSubcoreMesh(
        core_axis_name="c", subcore_axis_name="s",
        num_cores=NC, num_subcores=NS,
    )
    run = pl.kernel(
        _body,
        out_type=jax.ShapeDtypeStruct((BATCH, D), jnp.float32),
        mesh=mesh,
        scratch_types=[
            pltpu.VMEM((BPW,), jnp.int32),
            pltpu.VMEM((BPW,), jnp.int32),
            pltpu.VMEM((BPW,), jnp.int32),
            pltpu.VMEM((D,), jnp.float32),
            pltpu.VMEM((BPW, D), jnp.float32),
            pltpu.VMEM((BPW, D), jnp.float32),
            pltpu.SemaphoreType.DMA,
            pltpu.SemaphoreType.DMA,
            pltpu.SemaphoreType.DMA,
        ],
    )
    return run(input_data, unigram_probs, bigram_probs, tri2d)


# speedup vs baseline: 1.2787x; 1.0420x over previous
import jax
import jax.numpy as jnp
from jax import lax
from jax.experimental import pallas as pl
from jax.experimental.pallas import tpu as pltpu
from jax.experimental.pallas import tpu_sc as plsc

VOCAB = 512
SEQ = 50
BATCH = 1024
A0 = 1.0 / 100.0
A1 = 39.0 / 100.0
A2 = 6.0 / 10.0

NC = 2
NS = 16
L = 16
NW = NC * NS
BPW = BATCH // NW
D = VOCAB
NCHUNK = D // L
OUT_GROUPS = 4
ROWS_PER_GROUP = BPW // OUT_GROUPS


def _body(tokens_hbm, uni_hbm, bi_hbm, tri_hbm, out_hbm,
          prev_v, last_v, idx_v, uni_v, bi_v, tri_v,
          bi_sem, tri_sem, out_sem):
    wid = lax.axis_index("s") * NC + lax.axis_index("c")
    base = wid * BPW

    pltpu.sync_copy(tokens_hbm.at[SEQ - 2, pl.ds(base, BPW)], prev_v)
    pltpu.sync_copy(tokens_hbm.at[SEQ - 1, pl.ds(base, BPW)], last_v)

    for c in range(BPW // L):
        sl = pl.ds(c * L, L)
        idx_v[sl] = prev_v[sl] * VOCAB + last_v[sl]

    bi_copy = pltpu.make_async_copy(bi_hbm.at[last_v], bi_v, bi_sem)
    bi_copy.start()
    tri_copy = pltpu.make_async_copy(tri_hbm.at[idx_v], tri_v, tri_sem)
    tri_copy.start()

    with jax.named_scope("uni_stage"):
        pltpu.sync_copy(uni_hbm, uni_v)
        uni_c = [uni_v[pl.ds(c * L, L)] * A0 for c in range(NCHUNK)]

    with jax.named_scope("gather_wait"):
        bi_copy.wait()
        tri_copy.wait()

    def row(r, carry):
        for c in range(NCHUNK):
            sl = pl.ds(c * L, L)
            bi_v[r, sl] = uni_c[c] + A1 * bi_v[r, sl] + A2 * tri_v[r, sl]
        return carry

    out_copies = []
    for g in range(OUT_GROUPS):
        r0 = g * ROWS_PER_GROUP
        with jax.named_scope("blend"):
            lax.fori_loop(r0, r0 + ROWS_PER_GROUP, row, 0)
        cp = pltpu.make_async_copy(
            bi_v.at[pl.ds(r0, ROWS_PER_GROUP)],
            out_hbm.at[pl.ds(base + r0, ROWS_PER_GROUP)],
            out_sem,
        )
        cp.start()
        out_copies.append(cp)
    with jax.named_scope("out_drain"):
        for cp in out_copies:
            cp.wait()


@jax.jit
def kernel(input_data, unigram_probs, bigram_probs, trigram_probs):
    tri2d = trigram_probs.reshape(VOCAB * VOCAB, VOCAB)

    mesh = plsc.VectorSubcoreMesh(
        core_axis_name="c", subcore_axis_name="s",
        num_cores=NC, num_subcores=NS,
    )
    run = pl.kernel(
        _body,
        out_type=jax.ShapeDtypeStruct((BATCH, D), jnp.float32),
        mesh=mesh,
        scratch_types=[
            pltpu.VMEM((BPW,), jnp.int32),
            pltpu.VMEM((BPW,), jnp.int32),
            pltpu.VMEM((BPW,), jnp.int32),
            pltpu.VMEM((D,), jnp.float32),
            pltpu.VMEM((BPW, D), jnp.float32),
            pltpu.VMEM((BPW, D), jnp.float32),
            pltpu.SemaphoreType.DMA,
            pltpu.SemaphoreType.DMA,
            pltpu.SemaphoreType.DMA,
        ],
    )
    return run(input_data, unigram_probs, bigram_probs, tri2d)


# speedup vs baseline: 1.3355x; 1.0444x over previous
import jax
import jax.numpy as jnp
from jax import lax
from jax.experimental import pallas as pl
from jax.experimental.pallas import tpu as pltpu
from jax.experimental.pallas import tpu_sc as plsc

VOCAB = 512
SEQ = 50
BATCH = 1024
A0 = 1.0 / 100.0
A1 = 39.0 / 100.0
A2 = 6.0 / 10.0

NC = 2
NS = 16
L = 16
NW = NC * NS
BPW = BATCH // NW
D = VOCAB
NCHUNK = D // L
NQ = 4
RPQ = BPW // NQ


def _body(tokens_hbm, uni_hbm, bi_hbm, tri_hbm, out_hbm,
          prev_v, last_v, idx_v, uni_v, bi_v, tri_v,
          uni_sem, tok_sem, bi_sem0, bi_sem1, bi_sem2, bi_sem3,
          tri_sem0, tri_sem1, tri_sem2, tri_sem3, out_sem):
    wid = lax.axis_index("s") * NC + lax.axis_index("c")
    base = wid * BPW
    bi_sems = (bi_sem0, bi_sem1, bi_sem2, bi_sem3)
    tri_sems = (tri_sem0, tri_sem1, tri_sem2, tri_sem3)

    uni_copy = pltpu.make_async_copy(uni_hbm, uni_v, uni_sem)
    uni_copy.start()

    prev_copy = pltpu.make_async_copy(
        tokens_hbm.at[SEQ - 2, pl.ds(base, BPW)], prev_v, tok_sem)
    last_copy = pltpu.make_async_copy(
        tokens_hbm.at[SEQ - 1, pl.ds(base, BPW)], last_v, tok_sem)
    prev_copy.start()
    last_copy.start()
    prev_copy.wait()
    last_copy.wait()

    for c in range(BPW // L):
        sl = pl.ds(c * L, L)
        idx_v[sl] = prev_v[sl] * VOCAB + last_v[sl]

    copies = []
    for q in range(NQ):
        rows = pl.ds(q * RPQ, RPQ)
        cp_bi = pltpu.make_async_copy(
            bi_hbm.at[last_v.at[rows]], bi_v.at[rows], bi_sems[q])
        cp_tri = pltpu.make_async_copy(
            tri_hbm.at[idx_v.at[rows]], tri_v.at[rows], tri_sems[q])
        cp_bi.start()
        cp_tri.start()
        copies.append((cp_bi, cp_tri))

    uni_copy.wait()
    uni_c = [uni_v[pl.ds(c * L, L)] * A0 for c in range(NCHUNK)]

    def row(r, carry):
        for c in range(NCHUNK):
            sl = pl.ds(c * L, L)
            bi_v[r, sl] = uni_c[c] + A1 * bi_v[r, sl] + A2 * tri_v[r, sl]
        return carry

    out_copies = []
    for q, (cp_bi, cp_tri) in enumerate(copies):
        r0 = q * RPQ
        cp_bi.wait()
        cp_tri.wait()
        lax.fori_loop(r0, r0 + RPQ, row, 0)
        cp_out = pltpu.make_async_copy(
            bi_v.at[pl.ds(r0, RPQ)],
            out_hbm.at[pl.ds(base + r0, RPQ)],
            out_sem,
        )
        cp_out.start()
        out_copies.append(cp_out)
    for cp_out in out_copies:
        cp_out.wait()


@jax.jit
def kernel(input_data, unigram_probs, bigram_probs, trigram_probs):
    tri2d = trigram_probs.reshape(VOCAB * VOCAB, VOCAB)

    mesh = plsc.VectorSubcoreMesh(
        core_axis_name="c", subcore_axis_name="s",
        num_cores=NC, num_subcores=NS,
    )
    run = pl.kernel(
        _body,
        out_type=jax.ShapeDtypeStruct((BATCH, D), jnp.float32),
        mesh=mesh,
        scratch_types=[
            pltpu.VMEM((BPW,), jnp.int32),
            pltpu.VMEM((BPW,), jnp.int32),
            pltpu.VMEM((BPW,), jnp.int32),
            pltpu.VMEM((D,), jnp.float32),
            pltpu.VMEM((BPW, D), jnp.float32),
            pltpu.VMEM((BPW, D), jnp.float32),
        ] + [pltpu.SemaphoreType.DMA] * 11,
    )
    return run(input_data, unigram_probs, bigram_probs, tri2d)


# speedup vs baseline: 1.3567x; 1.0159x over previous
import jax
import jax.numpy as jnp
from jax import lax
from jax.experimental import pallas as pl
from jax.experimental.pallas import tpu as pltpu
from jax.experimental.pallas import tpu_sc as plsc

VOCAB = 512
SEQ = 50
BATCH = 1024
A0 = 1.0 / 100.0
A1 = 39.0 / 100.0
A2 = 6.0 / 10.0

NC = 2
NS = 16
L = 16
NW = NC * NS
BPW = BATCH // NW
D = VOCAB
NCHUNK = D // L
NQ = 2
RPQ = BPW // NQ


def _body(tokens_hbm, uni_hbm, bi_hbm, tri_hbm, out_hbm,
          prev_v, last_v, idx_v, uni_v, bi_v, tri_v,
          uni_sem, tok_sem, bi_sem0, bi_sem1,
          tri_sem0, tri_sem1, out_sem):
    wid = lax.axis_index("s") * NC + lax.axis_index("c")
    base = wid * BPW
    bi_sems = (bi_sem0, bi_sem1)
    tri_sems = (tri_sem0, tri_sem1)

    uni_copy = pltpu.make_async_copy(uni_hbm, uni_v, uni_sem)
    uni_copy.start()

    prev_copy = pltpu.make_async_copy(
        tokens_hbm.at[SEQ - 2, pl.ds(base, BPW)], prev_v, tok_sem)
    last_copy = pltpu.make_async_copy(
        tokens_hbm.at[SEQ - 1, pl.ds(base, BPW)], last_v, tok_sem)
    prev_copy.start()
    last_copy.start()
    prev_copy.wait()
    last_copy.wait()

    for c in range(BPW // L):
        sl = pl.ds(c * L, L)
        idx_v[sl] = prev_v[sl] * VOCAB + last_v[sl]

    copies = []
    for q in range(NQ):
        rows = pl.ds(q * RPQ, RPQ)
        cp_bi = pltpu.make_async_copy(
            bi_hbm.at[last_v.at[rows]], bi_v.at[rows], bi_sems[q])
        cp_tri = pltpu.make_async_copy(
            tri_hbm.at[idx_v.at[rows]], tri_v.at[rows], tri_sems[q])
        cp_bi.start()
        cp_tri.start()
        copies.append((cp_bi, cp_tri))

    uni_copy.wait()
    uni_c = [uni_v[pl.ds(c * L, L)] * A0 for c in range(NCHUNK)]

    def row(r, carry):
        for c in range(NCHUNK):
            sl = pl.ds(c * L, L)
            bi_v[r, sl] = uni_c[c] + A1 * bi_v[r, sl] + A2 * tri_v[r, sl]
        return carry

    out_copies = []
    for q, (cp_bi, cp_tri) in enumerate(copies):
        r0 = q * RPQ
        cp_bi.wait()
        cp_tri.wait()
        lax.fori_loop(r0, r0 + RPQ, row, 0)
        cp_out = pltpu.make_async_copy(
            bi_v.at[pl.ds(r0, RPQ)],
            out_hbm.at[pl.ds(base + r0, RPQ)],
            out_sem,
        )
        cp_out.start()
        out_copies.append(cp_out)
    for cp_out in out_copies:
        cp_out.wait()


@jax.jit
def kernel(input_data, unigram_probs, bigram_probs, trigram_probs):
    tri2d = trigram_probs.reshape(VOCAB * VOCAB, VOCAB)

    mesh = plsc.VectorSubcoreMesh(
        core_axis_name="c", subcore_axis_name="s",
        num_cores=NC, num_subcores=NS,
    )
    run = pl.kernel(
        _body,
        out_type=jax.ShapeDtypeStruct((BATCH, D), jnp.float32),
        mesh=mesh,
        scratch_types=[
            pltpu.VMEM((BPW,), jnp.int32),
            pltpu.VMEM((BPW,), jnp.int32),
            pltpu.VMEM((BPW,), jnp.int32),
            pltpu.VMEM((D,), jnp.float32),
            pltpu.VMEM((BPW, D), jnp.float32),
            pltpu.VMEM((BPW, D), jnp.float32),
        ] + [pltpu.SemaphoreType.DMA] * 7,
    )
    return run(input_data, unigram_probs, bigram_probs, tri2d)


# speedup vs baseline: 1.3576x; 1.0006x over previous
import jax
import jax.numpy as jnp
from jax import lax
from jax.experimental import pallas as pl
from jax.experimental.pallas import tpu as pltpu
from jax.experimental.pallas import tpu_sc as plsc

VOCAB = 512
SEQ = 50
BATCH = 1024
A0 = 1.0 / 100.0
A1 = 39.0 / 100.0
A2 = 6.0 / 10.0

NC = 2
NS = 16
L = 16
NW = NC * NS
BPW = BATCH // NW
D = VOCAB
NCHUNK = D // L
NQ = 1
RPQ = BPW // NQ


def _body(tokens_hbm, uni_hbm, bi_hbm, tri_hbm, out_hbm,
          prev_v, last_v, idx_v, uni_v, bi_v, tri_v,
          uni_sem, tok_sem, bi_sem0, tri_sem0, out_sem):
    wid = lax.axis_index("s") * NC + lax.axis_index("c")
    base = wid * BPW
    bi_sems = (bi_sem0,)
    tri_sems = (tri_sem0,)

    uni_copy = pltpu.make_async_copy(uni_hbm, uni_v, uni_sem)
    uni_copy.start()

    prev_copy = pltpu.make_async_copy(
        tokens_hbm.at[SEQ - 2, pl.ds(base, BPW)], prev_v, tok_sem)
    last_copy = pltpu.make_async_copy(
        tokens_hbm.at[SEQ - 1, pl.ds(base, BPW)], last_v, tok_sem)
    prev_copy.start()
    last_copy.start()
    prev_copy.wait()
    last_copy.wait()

    for c in range(BPW // L):
        sl = pl.ds(c * L, L)
        idx_v[sl] = prev_v[sl] * VOCAB + last_v[sl]

    copies = []
    for q in range(NQ):
        rows = pl.ds(q * RPQ, RPQ)
        cp_bi = pltpu.make_async_copy(
            bi_hbm.at[last_v.at[rows]], bi_v.at[rows], bi_sems[q])
        cp_tri = pltpu.make_async_copy(
            tri_hbm.at[idx_v.at[rows]], tri_v.at[rows], tri_sems[q])
        cp_bi.start()
        cp_tri.start()
        copies.append((cp_bi, cp_tri))

    uni_copy.wait()
    uni_c = [uni_v[pl.ds(c * L, L)] * A0 for c in range(NCHUNK)]

    def row(r, carry):
        for c in range(NCHUNK):
            sl = pl.ds(c * L, L)
            bi_v[r, sl] = uni_c[c] + A1 * bi_v[r, sl] + A2 * tri_v[r, sl]
        return carry

    out_copies = []
    for q, (cp_bi, cp_tri) in enumerate(copies):
        r0 = q * RPQ
        cp_bi.wait()
        cp_tri.wait()
        lax.fori_loop(r0, r0 + RPQ, row, 0)
        cp_out = pltpu.make_async_copy(
            bi_v.at[pl.ds(r0, RPQ)],
            out_hbm.at[pl.ds(base + r0, RPQ)],
            out_sem,
        )
        cp_out.start()
        out_copies.append(cp_out)
    for cp_out in out_copies:
        cp_out.wait()


@jax.jit
def kernel(input_data, unigram_probs, bigram_probs, trigram_probs):
    tri2d = trigram_probs.reshape(VOCAB * VOCAB, VOCAB)

    mesh = plsc.VectorSubcoreMesh(
        core_axis_name="c", subcore_axis_name="s",
        num_cores=NC, num_subcores=NS,
    )
    run = pl.kernel(
        _body,
        out_type=jax.ShapeDtypeStruct((BATCH, D), jnp.float32),
        mesh=mesh,
        scratch_types=[
            pltpu.VMEM((BPW,), jnp.int32),
            pltpu.VMEM((BPW,), jnp.int32),
            pltpu.VMEM((BPW,), jnp.int32),
            pltpu.VMEM((D,), jnp.float32),
            pltpu.VMEM((BPW, D), jnp.float32),
            pltpu.VMEM((BPW, D), jnp.float32),
        ] + [pltpu.SemaphoreType.DMA] * 5,
    )
    return run(input_data, unigram_probs, bigram_probs, tri2d)


# speedup vs baseline: 1.3675x; 1.0073x over previous
import jax
import jax.numpy as jnp
from jax import lax
from jax.experimental import pallas as pl
from jax.experimental.pallas import tpu as pltpu
from jax.experimental.pallas import tpu_sc as plsc

VOCAB = 512
SEQ = 50
BATCH = 1024
A0 = 1.0 / 100.0
A1 = 39.0 / 100.0
A2 = 6.0 / 10.0

NC = 2
NS = 16
L = 16
NW = NC * NS
BPW = BATCH // NW
D = VOCAB
NCHUNK = D // L
NQ = 1
RPQ = BPW // NQ


def _body(tokens_hbm, uni_hbm, bi_hbm, tri_hbm, out_hbm,
          prev_v, last_v, idx_v, uni_v, bi_v, tri_v,
          uni_sem, tok_sem, bi_sem0, tri_sem0, out_sem):
    wid = lax.axis_index("s") * NC + lax.axis_index("c")
    base = wid * BPW
    bi_sems = (bi_sem0,)
    tri_sems = (tri_sem0,)

    uni_copy = pltpu.make_async_copy(uni_hbm, uni_v, uni_sem)
    uni_copy.start()

    prev_copy = pltpu.make_async_copy(
        tokens_hbm.at[SEQ - 2, pl.ds(base, BPW)], prev_v, tok_sem)
    last_copy = pltpu.make_async_copy(
        tokens_hbm.at[SEQ - 1, pl.ds(base, BPW)], last_v, tok_sem)
    prev_copy.start()
    last_copy.start()
    prev_copy.wait()
    last_copy.wait()

    for c in range(BPW // L):
        sl = pl.ds(c * L, L)
        idx_v[sl] = prev_v[sl] * VOCAB + last_v[sl]

    copies = []
    for q in range(NQ):
        rows = pl.ds(q * RPQ, RPQ)
        cp_bi = pltpu.make_async_copy(
            bi_hbm.at[last_v.at[rows]], bi_v.at[rows], bi_sems[q])
        cp_tri = pltpu.make_async_copy(
            tri_hbm.at[idx_v.at[rows]], tri_v.at[rows], tri_sems[q])
        cp_bi.start()
        cp_tri.start()
        copies.append((cp_bi, cp_tri))

    uni_copy.wait()
    uni_c = [uni_v[pl.ds(c * L, L)] * A0 for c in range(NCHUNK)]

    def row(r, carry):
        for c in range(NCHUNK):
            sl = pl.ds(c * L, L)
            bi_v[r, sl] = uni_c[c] + A1 * bi_v[r, sl] + A2 * tri_v[r, sl]
        return carry

    copies[0][0].wait()
    copies[0][1].wait()
    out_copies = []
    for h in range(2):
        r0 = h * (BPW // 2)
        lax.fori_loop(r0, r0 + BPW // 2, row, 0)
        cp_out = pltpu.make_async_copy(
            bi_v.at[pl.ds(r0, BPW // 2)],
            out_hbm.at[pl.ds(base + r0, BPW // 2)],
            out_sem,
        )
        cp_out.start()
        out_copies.append(cp_out)
    for cp_out in out_copies:
        cp_out.wait()


@jax.jit
def kernel(input_data, unigram_probs, bigram_probs, trigram_probs):
    tri2d = trigram_probs.reshape(VOCAB * VOCAB, VOCAB)

    mesh = plsc.VectorSubcoreMesh(
        core_axis_name="c", subcore_axis_name="s",
        num_cores=NC, num_subcores=NS,
    )
    run = pl.kernel(
        _body,
        out_type=jax.ShapeDtypeStruct((BATCH, D), jnp.float32),
        mesh=mesh,
        scratch_types=[
            pltpu.VMEM((BPW,), jnp.int32),
            pltpu.VMEM((BPW,), jnp.int32),
            pltpu.VMEM((BPW,), jnp.int32),
            pltpu.VMEM((D,), jnp.float32),
            pltpu.VMEM((BPW, D), jnp.float32),
            pltpu.VMEM((BPW, D), jnp.float32),
        ] + [pltpu.SemaphoreType.DMA] * 5,
    )
    return run(input_data, unigram_probs, bigram_probs, tri2d)
